# baseline (speedup 1.0000x reference)
import jax
import jax.numpy as jnp
from jax import lax
from jax.experimental import pallas as pl
from jax.experimental.pallas import tpu as pltpu
from jax.experimental.pallas import tpu_sc as plsc

NC = 2
NS = 16
NW = NC * NS
LANES = 16

N_NODES = 100000
N_EDGES = 3200000

CPT = ((N_NODES + NS - 1) // NS + LANES - 1) // LANES * LANES
NPAD = NS * CPT
W = 2048
ROUNDS = N_EDGES // (NW * W)
TA = 1664
TA_BASE = ROUNDS * NW * W
TB = 128
TB_BASE = TA_BASE + NW * TA
TAILW = TA + TB
NBUF = 2


def _sc_body(x_hbm, eij_hbm, a_hbm, out_hbm, *refs):
    eij_v = refs[0:2]
    a_v = refs[2:4]
    c_v = refs[4:6]
    rc_v = refs[6:8]
    x_v, acc_sh = refs[8], refs[9]
    in_sem = refs[10:12]
    out_sem = refs[12:14]

    cid = lax.axis_index("c")
    sid = lax.axis_index("s")
    wid = cid * NS + sid

    def fire_in(w, b):
        wc = jnp.minimum(w, ROUNDS - 1)
        eb = (wc * NW + wid) * W
        pltpu.async_copy(eij_hbm.at[pl.ds(eb // 128, W // 128)], eij_v[b],
                         in_sem[b])
        pltpu.async_copy(a_hbm.at[pl.ds(eb, W)], a_v[b], in_sem[b])

    def wait_in(b):
        pltpu.make_async_copy(eij_hbm.at[pl.ds(0, W // 128)], eij_v[b],
                              in_sem[b]).wait()
        pltpu.make_async_copy(a_hbm.at[pl.ds(0, W)], a_v[b], in_sem[b]).wait()

    def compute(b, nvec):
        def mbody(j, _):
            s = pl.ds(pl.multiple_of(j * LANES, LANES), LANES)
            t = j // (128 // LANES)
            sl = pl.ds(pl.multiple_of((j % (128 // LANES)) * LANES, LANES),
                       LANES)
            xg = plsc.load_gather(x_v, [eij_v[b][t, 1, sl]])
            c_v[b][s] = a_v[b][s] * xg
            rc_v[b][s] = eij_v[b][t, 0, sl]
            return 0
        lax.fori_loop(0, nvec, mbody, 0)

    def fire_scatter(b):
        pltpu.async_copy(c_v[b], acc_sh.at[rc_v[b]], out_sem[b], add=True)

    def wait_scatter(b):
        pltpu.make_async_copy(c_v[b], acc_sh.at[rc_v[b]], out_sem[b]).wait()

    def zbody(i, _):
        c_v[0][pl.ds(pl.multiple_of(i * LANES, LANES), LANES)] = (
            jnp.zeros((LANES,), jnp.float32))
        return 0
    lax.fori_loop(0, W // LANES, zbody, 0)

    nbase = sid * CPT
    for off in range(0, CPT, W):
        sz = min(W, CPT - off)
        pltpu.sync_copy(c_v[0].at[pl.ds(0, sz)],
                        acc_sh.at[pl.ds(nbase + off, sz)])
    pltpu.sync_copy(x_hbm, x_v)
    for b in range(NBUF):
        fire_in(b, b)
    plsc.subcore_barrier()

    for w in range(NBUF):
        wait_in(w)
        compute(w, W // LANES)
        fire_scatter(w)
        fire_in(w + NBUF, w)

    def body(i, _):
        for b in range(NBUF):
            w = i * NBUF + b
            wait_in(b)
            wait_scatter(b)
            compute(b, W // LANES)
            fire_scatter(b)
            fire_in(w + NBUF, b)
        return 0
    lax.fori_loop(1, ROUNDS // NBUF, body, 0)

    for b in range(NBUF):
        wait_in(b)
        wait_scatter(b)

    ta = TA_BASE + wid * TA
    tb = TB_BASE + jnp.where(wid < 8, wid, 0) * TB
    pltpu.sync_copy(eij_hbm.at[pl.ds(ta // 128, TA // 128)],
                    eij_v[0].at[pl.ds(0, TA // 128)])
    pltpu.sync_copy(a_hbm.at[pl.ds(ta, TA)], a_v[0].at[pl.ds(0, TA)])
    pltpu.sync_copy(eij_hbm.at[pl.ds(tb // 128, TB // 128)],
                    eij_v[0].at[pl.ds(TA // 128, TB // 128)])
    pltpu.sync_copy(a_hbm.at[pl.ds(tb, TB)], a_v[0].at[pl.ds(TA, TB)])
    compute(0, TAILW // LANES)

    def czero(lo, nvec):
        def zb(j, _):
            c_v[0][pl.ds(pl.multiple_of(lo + j * LANES, LANES), LANES)] = (
                jnp.zeros((LANES,), jnp.float32))
            return 0
        lax.fori_loop(0, nvec, zb, 0)

    @pl.when(wid >= 8)
    def _():
        czero(TA, TB // LANES)
    czero(TAILW, (W - TAILW) // LANES)
    pltpu.sync_copy(c_v[0], acc_sh.at[rc_v[0]], add=True)

    plsc.subcore_barrier()
    for off in range(0, CPT, W):
        sz = min(W, CPT - off)
        pltpu.sync_copy(acc_sh.at[pl.ds(nbase + off, sz)],
                        c_v[0].at[pl.ds(0, sz)])
        pltpu.sync_copy(c_v[0].at[pl.ds(0, sz)],
                        out_hbm.at[pl.ds(cid * NPAD + nbase + off, sz)])


def _combine_body(p_ref, b_ref, o_ref):
    o_ref[...] = (b_ref[...] - p_ref[pl.ds(0, N_NODES)]
                  - p_ref[pl.ds(NPAD, N_NODES)])


@jax.jit
def kernel(vertex_attr, edgeij_pair, edge_attr):
    n = vertex_attr.shape[0]
    e = edgeij_pair.shape[1]
    eij = (edgeij_pair.astype(jnp.int32)
           .reshape(2, e // 128, 128).transpose(1, 0, 2))
    a = edge_attr.reshape(-1)
    b = vertex_attr[:, 0]
    x = vertex_attr[:, 1]

    mesh = plsc.VectorSubcoreMesh(core_axis_name="c", subcore_axis_name="s")
    partials = pl.kernel(
        _sc_body,
        out_type=jax.ShapeDtypeStruct((NC * NPAD,), jnp.float32),
        mesh=mesh,
        compiler_params=pltpu.CompilerParams(needs_layout_passes=False),
        scratch_types=(
            [pltpu.VMEM((W // 128, 2, 128), jnp.int32) for _ in range(2)]
            + [pltpu.VMEM((W,), jnp.float32) for _ in range(2)]
            + [pltpu.VMEM((W,), jnp.float32) for _ in range(2)]
            + [pltpu.VMEM((W,), jnp.int32) for _ in range(2)]
            + [
                pltpu.VMEM((N_NODES,), jnp.float32),
                pltpu.VMEM_SHARED((NPAD,), jnp.float32),
            ]
            + [pltpu.SemaphoreType.DMA for _ in range(4)]
        ),
    )(x, eij, a)

    r = pl.pallas_call(
        _combine_body,
        out_shape=jax.ShapeDtypeStruct((n,), jnp.float32),
    )(partials, b)
    return r.reshape(n, 1)


# speedup vs baseline: 1.7144x; 1.7144x over previous
import jax
import jax.numpy as jnp
from jax import lax
from jax.experimental import pallas as pl
from jax.experimental.pallas import tpu as pltpu
from jax.experimental.pallas import tpu_sc as plsc

NC = 2
NS = 16
NW = NC * NS
LANES = 16

N_NODES = 100000
N_EDGES = 3200000

CPT = ((N_NODES + NS - 1) // NS + LANES - 1) // LANES * LANES
NPAD = NS * CPT
W = 2048
ROUNDS = N_EDGES // (NW * W)
TA = 1664
TA_BASE = ROUNDS * NW * W
TB = 128
TB_BASE = TA_BASE + NW * TA
TAILW = TA + TB
NBUF = 2


def _sc_body(x_hbm, eij_hbm, a_hbm, out_hbm, *refs):
    eij_v = refs[0:2]
    a_v = refs[2:4]
    c_v = refs[4:6]
    rc_v = refs[6:8]
    x_v, acc_sh = refs[8], refs[9]
    in_sem = refs[10:12]
    out_sem = refs[12:14]

    cid = lax.axis_index("c")
    sid = lax.axis_index("s")
    wid = cid * NS + sid

    def fire_in(w, b):
        wc = jnp.minimum(w, ROUNDS - 1)
        eb = (wc * NW + wid) * W
        pltpu.async_copy(eij_hbm.at[pl.ds(eb // 128, W // 128)], eij_v[b],
                         in_sem[b])
        pltpu.async_copy(a_hbm.at[pl.ds(eb, W)], a_v[b], in_sem[b])

    def wait_in(b):
        pltpu.make_async_copy(eij_hbm.at[pl.ds(0, W // 128)], eij_v[b],
                              in_sem[b]).wait()
        pltpu.make_async_copy(a_hbm.at[pl.ds(0, W)], a_v[b], in_sem[b]).wait()

    def compute(b, ntiles):
        @plsc.parallel_loop(0, ntiles, unroll=4)
        def _(t):
            for l in range(128 // LANES):
                sl = pl.ds(l * LANES, LANES)
                s = pl.ds(pl.multiple_of(t * 128 + l * LANES, LANES), LANES)
                xg = plsc.load_gather(x_v, [eij_v[b][t, 1, sl]])
                c_v[b][s] = a_v[b][s] * xg
                rc_v[b][s] = eij_v[b][t, 0, sl]

    def fire_scatter(b):
        pltpu.async_copy(c_v[b], acc_sh.at[rc_v[b]], out_sem[b], add=True)

    def wait_scatter(b):
        pltpu.make_async_copy(c_v[b], acc_sh.at[rc_v[b]], out_sem[b]).wait()

    def zbody(i, _):
        c_v[0][pl.ds(pl.multiple_of(i * LANES, LANES), LANES)] = (
            jnp.zeros((LANES,), jnp.float32))
        return 0
    lax.fori_loop(0, W // LANES, zbody, 0)

    nbase = sid * CPT
    xcopy = pltpu.async_copy(x_hbm, x_v, out_sem[0])
    for b in range(NBUF):
        fire_in(b, b)
    for off in range(0, CPT, W):
        sz = min(W, CPT - off)
        pltpu.sync_copy(c_v[0].at[pl.ds(0, sz)],
                        acc_sh.at[pl.ds(nbase + off, sz)])
    xcopy.wait()
    plsc.subcore_barrier()

    for w in range(NBUF):
        wait_in(w)
        compute(w, W // 128)
        fire_scatter(w)
        fire_in(w + NBUF, w)

    def body(i, _):
        for b in range(NBUF):
            w = i * NBUF + b
            wait_in(b)
            wait_scatter(b)
            compute(b, W // 128)
            fire_scatter(b)
            fire_in(w + NBUF, b)
        return 0
    lax.fori_loop(1, ROUNDS // NBUF, body, 0)

    for b in range(NBUF):
        wait_in(b)
        wait_scatter(b)

    ta = TA_BASE + wid * TA
    tb = TB_BASE + jnp.where(wid < 8, wid, 0) * TB
    pltpu.sync_copy(eij_hbm.at[pl.ds(ta // 128, TA // 128)],
                    eij_v[0].at[pl.ds(0, TA // 128)])
    pltpu.sync_copy(a_hbm.at[pl.ds(ta, TA)], a_v[0].at[pl.ds(0, TA)])
    pltpu.sync_copy(eij_hbm.at[pl.ds(tb // 128, TB // 128)],
                    eij_v[0].at[pl.ds(TA // 128, TB // 128)])
    pltpu.sync_copy(a_hbm.at[pl.ds(tb, TB)], a_v[0].at[pl.ds(TA, TB)])
    compute(0, TAILW // 128)

    def czero(lo, nvec):
        def zb(j, _):
            c_v[0][pl.ds(pl.multiple_of(lo + j * LANES, LANES), LANES)] = (
                jnp.zeros((LANES,), jnp.float32))
            return 0
        lax.fori_loop(0, nvec, zb, 0)

    @pl.when(wid >= 8)
    def _():
        czero(TA, TB // LANES)
    czero(TAILW, (W - TAILW) // LANES)
    pltpu.sync_copy(c_v[0], acc_sh.at[rc_v[0]], add=True)

    plsc.subcore_barrier()
    for off in range(0, CPT, W):
        sz = min(W, CPT - off)
        pltpu.sync_copy(acc_sh.at[pl.ds(nbase + off, sz)],
                        c_v[0].at[pl.ds(0, sz)])
        pltpu.sync_copy(c_v[0].at[pl.ds(0, sz)],
                        out_hbm.at[pl.ds(cid * NPAD + nbase + off, sz)])


def _combine_body(p_ref, b_ref, o_ref):
    o_ref[...] = (b_ref[...] - p_ref[pl.ds(0, N_NODES)]
                  - p_ref[pl.ds(NPAD, N_NODES)])


@jax.jit
def kernel(vertex_attr, edgeij_pair, edge_attr):
    n = vertex_attr.shape[0]
    e = edgeij_pair.shape[1]
    assert n == N_NODES and e == N_EDGES, (n, e)
    eij = (edgeij_pair.astype(jnp.int32)
           .reshape(2, e // 128, 128).transpose(1, 0, 2))
    a = edge_attr.reshape(-1)
    b = vertex_attr[:, 0]
    x = vertex_attr[:, 1]

    mesh = plsc.VectorSubcoreMesh(core_axis_name="c", subcore_axis_name="s")
    partials = pl.kernel(
        _sc_body,
        out_type=jax.ShapeDtypeStruct((NC * NPAD,), jnp.float32),
        mesh=mesh,
        compiler_params=pltpu.CompilerParams(needs_layout_passes=False),
        scratch_types=(
            [pltpu.VMEM((W // 128, 2, 128), jnp.int32) for _ in range(2)]
            + [pltpu.VMEM((W,), jnp.float32) for _ in range(2)]
            + [pltpu.VMEM((W,), jnp.float32) for _ in range(2)]
            + [pltpu.VMEM((W,), jnp.int32) for _ in range(2)]
            + [
                pltpu.VMEM((N_NODES,), jnp.float32),
                pltpu.VMEM_SHARED((NPAD,), jnp.float32),
            ]
            + [pltpu.SemaphoreType.DMA for _ in range(4)]
        ),
    )(x, eij, a)

    r = pl.pallas_call(
        _combine_body,
        out_shape=jax.ShapeDtypeStruct((n,), jnp.float32),
    )(partials, b)
    return r.reshape(n, 1)
